# Initial kernel scaffold; baseline (speedup 1.0000x reference)
import functools

import jax
import jax.numpy as jnp
from jax import lax
from jax.experimental import pallas as pl
from jax.experimental.pallas import tpu as pltpu
from jax.experimental.pallas import tpu_sc as plsc

_N = 10000
_E = 320000
_NC = 2
_NS = 16
_CH = 80


def _make_agg(D):
    rows_per_tile = _N // _NS
    edges_per_tile = _E // (_NC * _NS)
    n_iter = edges_per_tile // _CH
    mesh = plsc.VectorSubcoreMesh(core_axis_name="c", subcore_axis_name="s")

    @functools.partial(
        pl.kernel,
        out_type=jax.ShapeDtypeStruct((_NC, _N, D), jnp.float32),
        mesh=mesh,
        scratch_types=[
            pltpu.VMEM((_CH,), jnp.int32),
            pltpu.VMEM((_CH,), jnp.int32),
            pltpu.VMEM((_CH, D), jnp.float32),
            pltpu.VMEM_SHARED((_N, D), jnp.float32),
            pltpu.SemaphoreType.DMA,
        ],
    )
    def agg(x_hbm, src_hbm, dst_hbm, out_hbm, src_v, dst_v, rows_v, acc, sem):
        c = lax.axis_index("c")
        s = lax.axis_index("s")
        r0 = s * rows_per_tile
        pltpu.sync_copy(x_hbm.at[pl.ds(r0, rows_per_tile)],
                        acc.at[pl.ds(r0, rows_per_tile)])
        plsc.subcore_barrier()

        ebase = c * (_E // _NC) + s * edges_per_tile

        def body(i, carry):
            off = ebase + i * _CH
            pltpu.sync_copy(src_hbm.at[pl.ds(off, _CH)], src_v)
            pltpu.sync_copy(dst_hbm.at[pl.ds(off, _CH)], dst_v)
            pltpu.async_copy(x_hbm.at[src_v], rows_v, sem).wait()
            pltpu.sync_copy(rows_v, acc.at[dst_v], add=True)
            return carry

        lax.fori_loop(0, n_iter, body, 0)
        plsc.subcore_barrier()
        pltpu.sync_copy(acc.at[pl.ds(r0, rows_per_tile)],
                        out_hbm.at[c, pl.ds(r0, rows_per_tile)])

    return agg


_agg_d128 = _make_agg(128)
_agg_d64 = _make_agg(64)


def _mlp1_body(x_ref, a_ref, w1_ref, b1_ref, g_ref, bt_ref, w2_ref, b2_ref,
               o_ref):
    h = a_ref[0] + a_ref[1] - x_ref[...]
    t = jnp.dot(h, w1_ref[...], preferred_element_type=jnp.float32)
    t = t + b1_ref[...]
    mu = jnp.mean(t, axis=0, keepdims=True)
    var = jnp.mean(jnp.square(t - mu), axis=0, keepdims=True)
    t = (t - mu) * lax.rsqrt(var + 1e-5) * g_ref[...] + bt_ref[...]
    t = jnp.maximum(t, 0.0)
    t = jnp.dot(t, w2_ref[...], preferred_element_type=jnp.float32)
    o_ref[...] = jnp.maximum(t + b2_ref[...], 0.0)


def _mlp2_body(x_ref, a_ref, w1_ref, b1_ref, g_ref, bt_ref, w2_ref, b2_ref,
               fcw_ref, fcb_ref, emb_ref, out_ref):
    h = a_ref[0] + a_ref[1] - x_ref[...]
    t = jnp.dot(h, w1_ref[...], preferred_element_type=jnp.float32)
    t = t + b1_ref[...]
    mu = jnp.mean(t, axis=0, keepdims=True)
    var = jnp.mean(jnp.square(t - mu), axis=0, keepdims=True)
    t = (t - mu) * lax.rsqrt(var + 1e-5) * g_ref[...] + bt_ref[...]
    t = jnp.maximum(t, 0.0)
    t = jnp.dot(t, w2_ref[...], preferred_element_type=jnp.float32)
    h2 = jnp.maximum(t + b2_ref[...], 0.0)
    emb_ref[...] = h2
    out_ref[...] = jnp.dot(h2, fcw_ref[...],
                           preferred_element_type=jnp.float32) + fcb_ref[...]


def kernel(x, edge_index, l1_w1, l1_b1, l1_bn_g, l1_bn_b, l1_w2, l1_b2,
           l2_w1, l2_b1, l2_bn_g, l2_bn_b, l2_w2, l2_b2, fc_w, fc_b):
    src = edge_index[0]
    dst = edge_index[1]

    agg1 = _agg_d128(x, src, dst)
    h1 = pl.pallas_call(
        _mlp1_body,
        out_shape=jax.ShapeDtypeStruct((_N, 64), jnp.float32),
    )(x, agg1, l1_w1, l1_b1, l1_bn_g, l1_bn_b, l1_w2, l1_b2)

    agg2 = _agg_d64(h1, src, dst)
    emb, out = pl.pallas_call(
        _mlp2_body,
        out_shape=(
            jax.ShapeDtypeStruct((_N, 32), jnp.float32),
            jax.ShapeDtypeStruct((_N, 64), jnp.float32),
        ),
    )(h1, agg2, l2_w1, l2_b1, l2_bn_g, l2_bn_b, l2_w2, l2_b2, fc_w, fc_b)

    return emb, out


# speedup vs baseline: 4.4358x; 4.4358x over previous
import functools

import jax
import jax.numpy as jnp
from jax import lax
from jax.experimental import pallas as pl
from jax.experimental.pallas import tpu as pltpu
from jax.experimental.pallas import tpu_sc as plsc

_N = 10000
_E = 320000
_NC = 2
_NS = 16
_CH = 80


def _make_agg(D):
    rows_per_tile = 624
    tail_rows = _N - _NS * rows_per_tile
    tail_r0 = _NS * rows_per_tile
    edges_per_tile = _E // (_NC * _NS)
    n_iter = edges_per_tile // _CH
    mesh = plsc.VectorSubcoreMesh(core_axis_name="c", subcore_axis_name="s")

    @functools.partial(
        pl.kernel,
        out_type=jax.ShapeDtypeStruct((_NC, _N, D), jnp.float32),
        mesh=mesh,
        scratch_types=[
            pltpu.VMEM((_CH,), jnp.int32),
            pltpu.VMEM((_CH,), jnp.int32),
            pltpu.VMEM((_CH, D), jnp.float32),
            pltpu.VMEM_SHARED((_N, D), jnp.float32),
            pltpu.SemaphoreType.DMA,
        ],
    )
    def agg(x_hbm, src_hbm, dst_hbm, out_hbm, src_v, dst_v, rows_v, acc, sem):
        c = lax.axis_index("c")
        s = lax.axis_index("s")
        r0 = pl.multiple_of(s * rows_per_tile, 8)
        pltpu.sync_copy(x_hbm.at[pl.ds(r0, rows_per_tile)],
                        acc.at[pl.ds(r0, rows_per_tile)])

        @pl.when(s == 0)
        def _seed_tail():
            pltpu.sync_copy(x_hbm.at[pl.ds(tail_r0, tail_rows)],
                            acc.at[pl.ds(tail_r0, tail_rows)])

        plsc.subcore_barrier()

        ebase = c * (_E // _NC) + s * edges_per_tile

        def body(i, carry):
            off = ebase + i * _CH
            pltpu.sync_copy(src_hbm.at[pl.ds(off, _CH)], src_v)
            pltpu.sync_copy(dst_hbm.at[pl.ds(off, _CH)], dst_v)
            pltpu.async_copy(x_hbm.at[src_v], rows_v, sem).wait()
            pltpu.sync_copy(rows_v, acc.at[dst_v], add=True)
            return carry

        lax.fori_loop(0, n_iter, body, 0)
        plsc.subcore_barrier()
        pltpu.sync_copy(acc.at[pl.ds(r0, rows_per_tile)],
                        out_hbm.at[c, pl.ds(r0, rows_per_tile)])

        @pl.when(s == 0)
        def _write_tail():
            pltpu.sync_copy(acc.at[pl.ds(tail_r0, tail_rows)],
                            out_hbm.at[c, pl.ds(tail_r0, tail_rows)])

    return agg


_agg_d128 = _make_agg(128)


def _mlp1_body(x_ref, a_ref, w1_ref, b1_ref, g_ref, bt_ref, w2_ref, b2_ref,
               o_ref):
    h = a_ref[0] + a_ref[1] - x_ref[...]
    t = jnp.dot(h, w1_ref[...], preferred_element_type=jnp.float32)
    t = t + b1_ref[...]
    mu = jnp.mean(t, axis=0, keepdims=True)
    var = jnp.mean(jnp.square(t - mu), axis=0, keepdims=True)
    t = (t - mu) * lax.rsqrt(var + 1e-5) * g_ref[...] + bt_ref[...]
    t = jnp.maximum(t, 0.0)
    t = jnp.dot(t, w2_ref[...], preferred_element_type=jnp.float32)
    o_ref[:, :64] = jnp.maximum(t + b2_ref[...], 0.0)
    o_ref[:, 64:] = jnp.zeros((_N, 64), jnp.float32)


def _mlp2_body(x_ref, a_ref, w1_ref, b1_ref, g_ref, bt_ref, w2_ref, b2_ref,
               fcw_ref, fcb_ref, emb_ref, out_ref):
    h = (a_ref[0] + a_ref[1] - x_ref[...])[:, :64]
    t = jnp.dot(h, w1_ref[...], preferred_element_type=jnp.float32)
    t = t + b1_ref[...]
    mu = jnp.mean(t, axis=0, keepdims=True)
    var = jnp.mean(jnp.square(t - mu), axis=0, keepdims=True)
    t = (t - mu) * lax.rsqrt(var + 1e-5) * g_ref[...] + bt_ref[...]
    t = jnp.maximum(t, 0.0)
    t = jnp.dot(t, w2_ref[...], preferred_element_type=jnp.float32)
    h2 = jnp.maximum(t + b2_ref[...], 0.0)
    emb_ref[...] = h2
    out_ref[...] = jnp.dot(h2, fcw_ref[...],
                           preferred_element_type=jnp.float32) + fcb_ref[...]


def kernel(x, edge_index, l1_w1, l1_b1, l1_bn_g, l1_bn_b, l1_w2, l1_b2,
           l2_w1, l2_b1, l2_bn_g, l2_bn_b, l2_w2, l2_b2, fc_w, fc_b):
    src = edge_index[0]
    dst = edge_index[1]

    agg1 = _agg_d128(x, src, dst)
    h1 = pl.pallas_call(
        _mlp1_body,
        out_shape=jax.ShapeDtypeStruct((_N, 128), jnp.float32),
    )(x, agg1, l1_w1, l1_b1, l1_bn_g, l1_bn_b, l1_w2, l1_b2)

    agg2 = _agg_d128(h1, src, dst)
    emb, out = pl.pallas_call(
        _mlp2_body,
        out_shape=(
            jax.ShapeDtypeStruct((_N, 32), jnp.float32),
            jax.ShapeDtypeStruct((_N, 64), jnp.float32),
        ),
    )(h1, agg2, l2_w1, l2_b1, l2_bn_g, l2_bn_b, l2_w2, l2_b2, fc_w, fc_b)

    return emb, out


# speedup vs baseline: 9.6059x; 2.1655x over previous
import functools

import jax
import jax.numpy as jnp
from jax import lax
from jax.experimental import pallas as pl
from jax.experimental.pallas import tpu as pltpu
from jax.experimental.pallas import tpu_sc as plsc

_N = 10000
_E = 320000
_NC = 2
_NS = 16
_CH = 80


def _make_agg(D):
    rows_per_tile = 624
    tail_rows = _N - _NS * rows_per_tile
    tail_r0 = _NS * rows_per_tile
    n_phase = 5
    n_chunk = 25
    mesh = plsc.VectorSubcoreMesh(core_axis_name="c", subcore_axis_name="s")

    @functools.partial(
        pl.kernel,
        out_type=jax.ShapeDtypeStruct((_NC, _N, D), jnp.float32),
        mesh=mesh,
        scratch_types=[
            pltpu.VMEM((2, n_chunk, _CH), jnp.int32),
            pltpu.VMEM((2, n_chunk, _CH), jnp.int32),
            pltpu.VMEM((_CH, D), jnp.float32),
            pltpu.VMEM((_CH, D), jnp.float32),
            pltpu.VMEM_SHARED((_N, D), jnp.float32),
            pltpu.SemaphoreType.DMA,
            pltpu.SemaphoreType.DMA,
            pltpu.SemaphoreType.DMA,
        ],
    )
    def agg(x_hbm, src_hbm, dst_hbm, out_hbm, srcs, dsts, buf0, buf1, acc,
            sem0, sem1, ssem):
        c = lax.axis_index("c")
        s = lax.axis_index("s")
        w = c * _NS + s
        r0 = pl.multiple_of(s * rows_per_tile, 8)
        pltpu.sync_copy(src_hbm.at[w, 0], srcs.at[0])
        pltpu.sync_copy(dst_hbm.at[w, 0], dsts.at[0])
        pltpu.sync_copy(x_hbm.at[pl.ds(r0, rows_per_tile)],
                        acc.at[pl.ds(r0, rows_per_tile)])

        @pl.when(s == 0)
        def _seed_tail():
            pltpu.sync_copy(x_hbm.at[pl.ds(tail_r0, tail_rows)],
                            acc.at[pl.ds(tail_r0, tail_rows)])

        plsc.subcore_barrier()

        for p in range(n_phase):
            pb = p % 2
            if p + 1 < n_phase:
                pltpu.async_copy(src_hbm.at[w, p + 1], srcs.at[1 - pb], ssem)
                pltpu.async_copy(dst_hbm.at[w, p + 1], dsts.at[1 - pb], ssem)

            pltpu.async_copy(x_hbm.at[srcs.at[pb, 0]], buf0, sem0)

            def body(i, carry, pb=pb):
                j = 2 * i
                pltpu.async_copy(x_hbm.at[srcs.at[pb, j + 1]], buf1, sem1)
                pltpu.make_async_copy(x_hbm.at[srcs.at[pb, j]], buf0,
                                      sem0).wait()
                pltpu.sync_copy(buf0, acc.at[dsts.at[pb, j]], add=True)
                pltpu.async_copy(x_hbm.at[srcs.at[pb, j + 2]], buf0, sem0)
                pltpu.make_async_copy(x_hbm.at[srcs.at[pb, j + 1]], buf1,
                                      sem1).wait()
                pltpu.sync_copy(buf1, acc.at[dsts.at[pb, j + 1]], add=True)
                return carry

            lax.fori_loop(0, (n_chunk - 1) // 2, body, 0)
            pltpu.make_async_copy(x_hbm.at[srcs.at[pb, n_chunk - 1]], buf0,
                                  sem0).wait()
            pltpu.sync_copy(buf0, acc.at[dsts.at[pb, n_chunk - 1]], add=True)

            if p + 1 < n_phase:
                pltpu.make_async_copy(src_hbm.at[w, p + 1], srcs.at[1 - pb],
                                      ssem).wait()
                pltpu.make_async_copy(dst_hbm.at[w, p + 1], dsts.at[1 - pb],
                                      ssem).wait()

        plsc.subcore_barrier()
        pltpu.sync_copy(acc.at[pl.ds(r0, rows_per_tile)],
                        out_hbm.at[c, pl.ds(r0, rows_per_tile)])

        @pl.when(s == 0)
        def _write_tail():
            pltpu.sync_copy(acc.at[pl.ds(tail_r0, tail_rows)],
                            out_hbm.at[c, pl.ds(tail_r0, tail_rows)])

    return agg


_agg_d128 = _make_agg(128)


def _mlp1_body(x_ref, a_ref, w1_ref, b1_ref, g_ref, bt_ref, w2_ref, b2_ref,
               o_ref):
    h = a_ref[0] + a_ref[1] - x_ref[...]
    t = jnp.dot(h, w1_ref[...], preferred_element_type=jnp.float32)
    t = t + b1_ref[...]
    mu = jnp.mean(t, axis=0, keepdims=True)
    var = jnp.mean(jnp.square(t - mu), axis=0, keepdims=True)
    t = (t - mu) * lax.rsqrt(var + 1e-5) * g_ref[...] + bt_ref[...]
    t = jnp.maximum(t, 0.0)
    t = jnp.dot(t, w2_ref[...], preferred_element_type=jnp.float32)
    o_ref[:, :64] = jnp.maximum(t + b2_ref[...], 0.0)
    o_ref[:, 64:] = jnp.zeros((_N, 64), jnp.float32)


def _mlp2_body(x_ref, a_ref, w1_ref, b1_ref, g_ref, bt_ref, w2_ref, b2_ref,
               fcw_ref, fcb_ref, emb_ref, out_ref):
    h = (a_ref[0] + a_ref[1] - x_ref[...])[:, :64]
    t = jnp.dot(h, w1_ref[...], preferred_element_type=jnp.float32)
    t = t + b1_ref[...]
    mu = jnp.mean(t, axis=0, keepdims=True)
    var = jnp.mean(jnp.square(t - mu), axis=0, keepdims=True)
    t = (t - mu) * lax.rsqrt(var + 1e-5) * g_ref[...] + bt_ref[...]
    t = jnp.maximum(t, 0.0)
    t = jnp.dot(t, w2_ref[...], preferred_element_type=jnp.float32)
    h2 = jnp.maximum(t + b2_ref[...], 0.0)
    emb_ref[...] = h2
    out_ref[...] = jnp.dot(h2, fcw_ref[...],
                           preferred_element_type=jnp.float32) + fcb_ref[...]


def kernel(x, edge_index, l1_w1, l1_b1, l1_bn_g, l1_bn_b, l1_w2, l1_b2,
           l2_w1, l2_b1, l2_bn_g, l2_bn_b, l2_w2, l2_b2, fc_w, fc_b):
    src = edge_index[0].reshape(_NC * _NS, 5, 25, _CH)
    dst = edge_index[1].reshape(_NC * _NS, 5, 25, _CH)

    agg1 = _agg_d128(x, src, dst)
    h1 = pl.pallas_call(
        _mlp1_body,
        out_shape=jax.ShapeDtypeStruct((_N, 128), jnp.float32),
    )(x, agg1, l1_w1, l1_b1, l1_bn_g, l1_bn_b, l1_w2, l1_b2)

    agg2 = _agg_d128(h1, src, dst)
    emb, out = pl.pallas_call(
        _mlp2_body,
        out_shape=(
            jax.ShapeDtypeStruct((_N, 32), jnp.float32),
            jax.ShapeDtypeStruct((_N, 64), jnp.float32),
        ),
    )(h1, agg2, l2_w1, l2_b1, l2_bn_g, l2_bn_b, l2_w2, l2_b2, fc_w, fc_b)

    return emb, out
